# Initial kernel scaffold; baseline (speedup 1.0000x reference)
import jax
import jax.numpy as jnp
from jax.experimental import pallas as pl


def kernel(x, edge_index, edge_attr, c1_Wl, c1_Wr, c1_We, c1_att, c1_b, c2_Wl, c2_Wr, c2_We, c2_att, c2_b, lin_W, lin_b):
    raise NotImplementedError("write your pallas kernel here")


# speedup vs baseline: 6.4943x; 6.4943x over previous
import functools

import jax
import jax.numpy as jnp
from jax import lax
from jax.experimental import pallas as pl
from jax.experimental.pallas import tpu as pltpu
from jax.experimental.pallas import tpu_sc as plsc

N_NODES = 10000
E_EDGES = 320000
D = 128
D_OUT = 64
NC, NS, L = 2, 16, 16
NW = NC * NS
EPT = E_EDGES // NW
CH = 80
NCHUNK = EPT // CH
ZROWS = 40
NSUB = D // L

_DN = lax.GatherDimensionNumbers(
    offset_dims=(), collapsed_slice_dims=(0,), start_index_map=(0,))


def _mm2_body(x_ref, wa_ref, wb_ref, oa_ref, ob_ref):
    x = x_ref[...]
    oa_ref[...] = jnp.dot(x, wa_ref[...], preferred_element_type=jnp.float32)
    ob_ref[...] = jnp.dot(x, wb_ref[...], preferred_element_type=jnp.float32)


def _mm2(x, wa, wb, blk):
    n, k = x.shape
    return pl.pallas_call(
        _mm2_body,
        grid=(n // blk,),
        in_specs=[
            pl.BlockSpec((blk, k), lambda i: (i, 0)),
            pl.BlockSpec(wa.shape, lambda i: (0, 0)),
            pl.BlockSpec(wb.shape, lambda i: (0, 0)),
        ],
        out_specs=[
            pl.BlockSpec((blk, wa.shape[1]), lambda i: (i, 0)),
            pl.BlockSpec((blk, wb.shape[1]), lambda i: (i, 0)),
        ],
        out_shape=[
            jax.ShapeDtypeStruct((n, wa.shape[1]), jnp.float32),
            jax.ShapeDtypeStruct((n, wb.shape[1]), jnp.float32),
        ],
    )(x, wa, wb)


def _merge_body(acc_ref, den_ref, b_ref, wl_ref, wr_ref, xl_ref, xr_ref):
    num = acc_ref[0] + acc_ref[1]
    i = pl.program_id(0)
    den = (den_ref[0, i, :] + den_ref[1, i, :])[:, None]
    h = num / (den + 1e-16) + b_ref[...]
    h = jnp.where(h > 0, h, jnp.exp(jnp.minimum(h, 0.0)) - 1.0)
    xl_ref[...] = jnp.dot(h, wl_ref[...], preferred_element_type=jnp.float32)
    xr_ref[...] = jnp.dot(h, wr_ref[...], preferred_element_type=jnp.float32)


def _merge(acc, den, b, wl, wr, blk=1000):
    return pl.pallas_call(
        _merge_body,
        grid=(N_NODES // blk,),
        in_specs=[
            pl.BlockSpec((NC, blk, D), lambda i: (0, i, 0)),
            pl.BlockSpec((NC, N_NODES // blk, blk), lambda i: (0, 0, 0)),
            pl.BlockSpec((1, D), lambda i: (0, 0)),
            pl.BlockSpec(wl.shape, lambda i: (0, 0)),
            pl.BlockSpec(wr.shape, lambda i: (0, 0)),
        ],
        out_specs=[
            pl.BlockSpec((blk, D), lambda i: (i, 0)),
            pl.BlockSpec((blk, D), lambda i: (i, 0)),
        ],
        out_shape=[
            jax.ShapeDtypeStruct((N_NODES, D), jnp.float32),
            jax.ShapeDtypeStruct((N_NODES, D), jnp.float32),
        ],
    )(acc, den.reshape(NC, N_NODES // blk, blk), b, wl, wr)


def _final_body(acc_ref, den_ref, b_ref, w_ref, lb_ref, o_ref):
    num = acc_ref[0] + acc_ref[1]
    i = pl.program_id(0)
    den = (den_ref[0, i, :] + den_ref[1, i, :])[:, None]
    h = num / (den + 1e-16) + b_ref[...]
    o_ref[...] = (
        jnp.dot(h, w_ref[...], preferred_element_type=jnp.float32) + lb_ref[...]
    )


def _final(acc, den, b, w, lb, blk=1000):
    return pl.pallas_call(
        _final_body,
        grid=(N_NODES // blk,),
        in_specs=[
            pl.BlockSpec((NC, blk, D), lambda i: (0, i, 0)),
            pl.BlockSpec((NC, N_NODES // blk, blk), lambda i: (0, 0, 0)),
            pl.BlockSpec((1, D), lambda i: (0, 0)),
            pl.BlockSpec(w.shape, lambda i: (0, 0)),
            pl.BlockSpec((1, D_OUT), lambda i: (0, 0)),
        ],
        out_specs=pl.BlockSpec((blk, D_OUT), lambda i: (i, 0)),
        out_shape=jax.ShapeDtypeStruct((N_NODES, D_OUT), jnp.float32),
    )(acc, den.reshape(NC, N_NODES // blk, blk), b, w, lb)


@functools.cache
def _edge_pass():
    mesh = plsc.VectorSubcoreMesh(core_axis_name="c", subcore_axis_name="s")

    @functools.partial(
        pl.kernel,
        out_type=(jax.ShapeDtypeStruct((NC, N_NODES, D), jnp.float32),
                  jax.ShapeDtypeStruct((NC, N_NODES), jnp.float32)),
        mesh=mesh,
        scratch_types=[
            pltpu.VMEM((8, CH), jnp.int32),
            pltpu.VMEM((8, CH), jnp.int32),
            pltpu.VMEM((CH, D), jnp.float32),
            pltpu.VMEM((CH, D), jnp.float32),
            pltpu.VMEM((CH, D), jnp.float32),
            pltpu.VMEM((D,), jnp.float32),
            pltpu.VMEM((ZROWS, D), jnp.float32),
            pltpu.VMEM((1024,), jnp.float32),
            pltpu.VMEM((CH,), jnp.float32),
            pltpu.VMEM_SHARED((N_NODES, D), jnp.float32),
            pltpu.VMEM_SHARED((N_NODES,), jnp.float32),
            pltpu.SemaphoreType.DMA,
            pltpu.SemaphoreType.DMA,
            pltpu.SemaphoreType.DMA,
        ],
    )
    def edge_pass(xl_hbm, xr_hbm, we_hbm, src_hbm, dst_hbm, att_hbm,
                  acc_hbm, den_hbm,
                  rs_v, rd_v, we_v, xl_v, xr_v, att_v, z_v, zd_v, ex_v,
                  acc_sh, den_sh, sem0, sem1, sem2):
        cid = lax.axis_index("c")
        sid = lax.axis_index("s")
        wid = sid * NC + cid
        base = wid * EPT

        pltpu.sync_copy(att_hbm, att_v)

        zv = jnp.zeros((L,), jnp.float32)

        def zrow(i, c):
            z_v[i // NSUB, pl.ds((i % NSUB) * L, L)] = zv
            return c
        lax.fori_loop(0, ZROWS * NSUB, zrow, 0)

        def dzero(i, c):
            zd_v[pl.ds(i * L, L)] = zv
            return c
        lax.fori_loop(0, 1024 // L, dzero, 0)

        @pl.when(sid < N_NODES // 1000)
        def _():
            off = pl.multiple_of(sid * 1000, 1000)
            pltpu.sync_copy(zd_v.at[pl.ds(0, 1000)], den_sh.at[pl.ds(off, 1000)])

        def zcopy(t, c):
            m = t * NS + sid
            off = pl.multiple_of(m * ZROWS, ZROWS)
            pltpu.sync_copy(z_v, acc_sh.at[pl.ds(off, ZROWS)])
            return c
        nt = N_NODES // ZROWS // NS
        lax.fori_loop(0, nt, zcopy, 0)

        @pl.when(sid < (N_NODES // ZROWS) % NS)
        def _():
            m = nt * NS + sid
            off = pl.multiple_of(m * ZROWS, ZROWS)
            pltpu.sync_copy(z_v, acc_sh.at[pl.ds(off, ZROWS)])
        plsc.subcore_barrier()

        att_k = [att_v[pl.ds(kk * L, L)] for kk in range(NSUB)]
        io = lax.iota(jnp.int32, L)

        def chunk(j, carry):
            pltpu.sync_copy(src_hbm.at[wid, j], rs_v.at[0])
            pltpu.sync_copy(dst_hbm.at[wid, j], rd_v.at[0])
            d0 = pltpu.async_copy(
                we_hbm.at[pl.ds(pl.multiple_of(base + j * CH, CH), CH)],
                we_v, sem0)
            d1 = pltpu.async_copy(xl_hbm.at[rd_v.at[0]], xl_v, sem1)
            d2 = pltpu.async_copy(xr_hbm.at[rs_v.at[0]], xr_v, sem2)
            d0.wait()
            d1.wait()
            d2.wait()

            def edge(e, exlane):
                acc = jnp.zeros((L,), jnp.float32)
                xr_regs = []
                for kk in range(NSUB):
                    sl = pl.ds(kk * L, L)
                    r = xr_v[e, sl]
                    t = xl_v[e, sl] + we_v[e, sl] + r
                    lk = jnp.maximum(t, t * 0.2)
                    acc = acc + lk * att_k[kk]
                    xr_regs.append(r)
                for sft in (1, 2, 4, 8):
                    acc = acc + lax.gather(
                        acc, (io ^ sft)[:, None], dimension_numbers=_DN,
                        slice_sizes=(1,),
                        mode=lax.GatherScatterMode.PROMISE_IN_BOUNDS)
                exv = jnp.exp(acc)
                for kk in range(NSUB):
                    we_v[e, pl.ds(kk * L, L)] = xr_regs[kk] * exv
                exlane = jnp.where(io == e % L, exv, exlane)
                ex_v[pl.ds(pl.multiple_of(e // L * L, L), L)] = exlane
                return exlane

            lax.fori_loop(0, CH, edge, jnp.zeros((L,), jnp.float32))

            pltpu.sync_copy(we_v, acc_sh.at[rd_v.at[0]], add=True)
            pltpu.sync_copy(ex_v, den_sh.at[rd_v.at[0]], add=True)
            return carry

        lax.fori_loop(0, NCHUNK, chunk, 0)

        plsc.subcore_barrier()

        @pl.when(sid == 0)
        def _():
            pltpu.sync_copy(acc_sh, acc_hbm.at[cid])
            pltpu.sync_copy(den_sh, den_hbm.at[cid])

    return edge_pass


def kernel(x, edge_index, edge_attr, c1_Wl, c1_Wr, c1_We, c1_att, c1_b,
           c2_Wl, c2_Wr, c2_We, c2_att, c2_b, lin_W, lin_b):
    src = edge_index[0].reshape(NW, NCHUNK, CH)
    dst = edge_index[1].reshape(NW, NCHUNK, CH)

    xl1, xr1 = _mm2(x, c1_Wl, c1_Wr, 1000)
    we1, we2 = _mm2(edge_attr, c1_We, c2_We, 2000)

    ep = _edge_pass()
    acc1, den1 = ep(xl1, xr1, we1, src, dst, c1_att)
    xl2, xr2 = _merge(acc1, den1, c1_b.reshape(1, D), c2_Wl, c2_Wr)
    acc2, den2 = ep(xl2, xr2, we2, src, dst, c2_att)
    return _final(acc2, den2, c2_b.reshape(1, D), lin_W, lin_b.reshape(1, D_OUT))


# speedup vs baseline: 6.7656x; 1.0418x over previous
import functools

import jax
import jax.numpy as jnp
from jax import lax
from jax.experimental import pallas as pl
from jax.experimental.pallas import tpu as pltpu
from jax.experimental.pallas import tpu_sc as plsc

N_NODES = 10000
E_EDGES = 320000
D = 128
D_OUT = 64
NC, NS, L = 2, 16, 16
NW = NC * NS
EPT = E_EDGES // NW
CH = 40
NCHUNK = EPT // CH
SUP = 10
NSUP = NCHUNK // SUP
ZROWS = 16
NSUB = D // L

_DN = lax.GatherDimensionNumbers(
    offset_dims=(), collapsed_slice_dims=(0,), start_index_map=(0,))


def _mm2_body(x_ref, wa_ref, wb_ref, oa_ref, ob_ref):
    x = x_ref[...]
    oa_ref[...] = jnp.dot(x, wa_ref[...], preferred_element_type=jnp.float32)
    ob_ref[...] = jnp.dot(x, wb_ref[...], preferred_element_type=jnp.float32)


def _mm2(x, wa, wb, blk):
    n, k = x.shape
    return pl.pallas_call(
        _mm2_body,
        grid=(n // blk,),
        in_specs=[
            pl.BlockSpec((blk, k), lambda i: (i, 0)),
            pl.BlockSpec(wa.shape, lambda i: (0, 0)),
            pl.BlockSpec(wb.shape, lambda i: (0, 0)),
        ],
        out_specs=[
            pl.BlockSpec((blk, wa.shape[1]), lambda i: (i, 0)),
            pl.BlockSpec((blk, wb.shape[1]), lambda i: (i, 0)),
        ],
        out_shape=[
            jax.ShapeDtypeStruct((n, wa.shape[1]), jnp.float32),
            jax.ShapeDtypeStruct((n, wb.shape[1]), jnp.float32),
        ],
    )(x, wa, wb)


def _merge_body(acc_ref, den_ref, b_ref, wl_ref, wr_ref, xl_ref, xr_ref):
    num = acc_ref[0] + acc_ref[1]
    i = pl.program_id(0)
    den = (den_ref[0, i, :] + den_ref[1, i, :])[:, None]
    h = num / (den + 1e-16) + b_ref[...]
    h = jnp.where(h > 0, h, jnp.exp(jnp.minimum(h, 0.0)) - 1.0)
    xl_ref[...] = jnp.dot(h, wl_ref[...], preferred_element_type=jnp.float32)
    xr_ref[...] = jnp.dot(h, wr_ref[...], preferred_element_type=jnp.float32)


def _merge(acc, den, b, wl, wr, blk=1000):
    return pl.pallas_call(
        _merge_body,
        grid=(N_NODES // blk,),
        in_specs=[
            pl.BlockSpec((NC, blk, D), lambda i: (0, i, 0)),
            pl.BlockSpec((NC, N_NODES // blk, blk), lambda i: (0, 0, 0)),
            pl.BlockSpec((1, D), lambda i: (0, 0)),
            pl.BlockSpec(wl.shape, lambda i: (0, 0)),
            pl.BlockSpec(wr.shape, lambda i: (0, 0)),
        ],
        out_specs=[
            pl.BlockSpec((blk, D), lambda i: (i, 0)),
            pl.BlockSpec((blk, D), lambda i: (i, 0)),
        ],
        out_shape=[
            jax.ShapeDtypeStruct((N_NODES, D), jnp.float32),
            jax.ShapeDtypeStruct((N_NODES, D), jnp.float32),
        ],
    )(acc, den.reshape(NC, N_NODES // blk, blk), b, wl, wr)


def _final_body(acc_ref, den_ref, b_ref, w_ref, lb_ref, o_ref):
    num = acc_ref[0] + acc_ref[1]
    i = pl.program_id(0)
    den = (den_ref[0, i, :] + den_ref[1, i, :])[:, None]
    h = num / (den + 1e-16) + b_ref[...]
    o_ref[...] = (
        jnp.dot(h, w_ref[...], preferred_element_type=jnp.float32) + lb_ref[...]
    )


def _final(acc, den, b, w, lb, blk=1000):
    return pl.pallas_call(
        _final_body,
        grid=(N_NODES // blk,),
        in_specs=[
            pl.BlockSpec((NC, blk, D), lambda i: (0, i, 0)),
            pl.BlockSpec((NC, N_NODES // blk, blk), lambda i: (0, 0, 0)),
            pl.BlockSpec((1, D), lambda i: (0, 0)),
            pl.BlockSpec(w.shape, lambda i: (0, 0)),
            pl.BlockSpec((1, D_OUT), lambda i: (0, 0)),
        ],
        out_specs=pl.BlockSpec((blk, D_OUT), lambda i: (i, 0)),
        out_shape=jax.ShapeDtypeStruct((N_NODES, D_OUT), jnp.float32),
    )(acc, den.reshape(NC, N_NODES // blk, blk), b, w, lb)


@functools.cache
def _edge_pass():
    mesh = plsc.VectorSubcoreMesh(core_axis_name="c", subcore_axis_name="s")

    @functools.partial(
        pl.kernel,
        out_type=(jax.ShapeDtypeStruct((NC, N_NODES, D), jnp.float32),
                  jax.ShapeDtypeStruct((NC, N_NODES), jnp.float32)),
        mesh=mesh,
        scratch_types=[
            pltpu.VMEM((SUP, CH), jnp.int32),
            pltpu.VMEM((SUP, CH), jnp.int32),
            pltpu.VMEM((2, CH, D), jnp.float32),
            pltpu.VMEM((2, CH, D), jnp.float32),
            pltpu.VMEM((2, CH, D), jnp.float32),
            pltpu.VMEM((D,), jnp.float32),
            pltpu.VMEM((ZROWS, D), jnp.float32),
            pltpu.VMEM((1024,), jnp.float32),
            pltpu.VMEM((96,), jnp.float32),
            pltpu.VMEM_SHARED((N_NODES, D), jnp.float32),
            pltpu.VMEM_SHARED((N_NODES,), jnp.float32),
            pltpu.SemaphoreType.DMA,
            pltpu.SemaphoreType.DMA,
            pltpu.SemaphoreType.DMA,
            pltpu.SemaphoreType.DMA,
        ],
    )
    def edge_pass(xl_hbm, xr_hbm, we_hbm, src_hbm, dst_hbm, att_hbm,
                  acc_hbm, den_hbm,
                  src_sup, dst_sup, we_v, xl_v, xr_v, att_v, z_v, zd_v, ex_v,
                  acc_sh, den_sh, sem_g0, sem_g1, sem_s0, sem_s1):
        cid = lax.axis_index("c")
        sid = lax.axis_index("s")
        wid = sid * NC + cid
        base = wid * EPT

        pltpu.sync_copy(att_hbm, att_v)

        zv = jnp.zeros((L,), jnp.float32)

        def zrow(i, c):
            z_v[i // NSUB, pl.ds((i % NSUB) * L, L)] = zv
            return c
        lax.fori_loop(0, ZROWS * NSUB, zrow, 0)

        def dzero(i, c):
            zd_v[pl.ds(i * L, L)] = zv
            return c
        lax.fori_loop(0, 1024 // L, dzero, 0)

        @pl.when(sid < N_NODES // 1000)
        def _():
            off = pl.multiple_of(sid * 1000, 1000)
            pltpu.sync_copy(zd_v.at[pl.ds(0, 1000)], den_sh.at[pl.ds(off, 1000)])

        nzc = N_NODES // ZROWS // NS
        zrem = (N_NODES // ZROWS) % NS

        def zfire(t, c):
            m = t * NS + sid
            off = pl.multiple_of(m * ZROWS, ZROWS)
            pltpu.make_async_copy(z_v, acc_sh.at[pl.ds(off, ZROWS)],
                                  sem_g0).start()
            return c
        lax.fori_loop(0, nzc, zfire, 0)

        @pl.when(sid < zrem)
        def _():
            m = nzc * NS + sid
            off = pl.multiple_of(m * ZROWS, ZROWS)
            pltpu.make_async_copy(z_v, acc_sh.at[pl.ds(off, ZROWS)],
                                  sem_g0).start()

        def zdrain(t, c):
            pltpu.make_async_copy(
                z_v, acc_sh.at[pl.ds(pl.multiple_of(0, ZROWS), ZROWS)],
                sem_g0).wait()
            return c
        lax.fori_loop(0, nzc, zdrain, 0)

        @pl.when(sid < zrem)
        def _():
            pltpu.make_async_copy(
                z_v, acc_sh.at[pl.ds(pl.multiple_of(0, ZROWS), ZROWS)],
                sem_g0).wait()
        plsc.subcore_barrier()

        att_k = [att_v[pl.ds(kk * L, L)] for kk in range(NSUB)]
        io = lax.iota(jnp.int32, L)
        sem_g = [sem_g0, sem_g1]
        sem_s = [sem_s0, sem_s1]

        def issue_gathers(j, pp):
            jj = j % SUP
            off = pl.multiple_of(base + j * CH, CH)
            for b in (0, 1):
                @pl.when(pp == b)
                def _():
                    pltpu.make_async_copy(we_hbm.at[pl.ds(off, CH)],
                                          we_v.at[b], sem_g[b]).start()
                    pltpu.make_async_copy(xl_hbm.at[dst_sup.at[jj]],
                                          xl_v.at[b], sem_g[b]).start()
                    pltpu.make_async_copy(xr_hbm.at[src_sup.at[jj]],
                                          xr_v.at[b], sem_g[b]).start()

        def wait_gathers(pp):
            for b in (0, 1):
                @pl.when(pp == b)
                def _():
                    off0 = pl.multiple_of(0, CH)
                    pltpu.make_async_copy(we_hbm.at[pl.ds(off0, CH)],
                                          we_v.at[b], sem_g[b]).wait()
                    pltpu.make_async_copy(xl_hbm.at[dst_sup.at[0]],
                                          xl_v.at[b], sem_g[b]).wait()
                    pltpu.make_async_copy(xr_hbm.at[src_sup.at[0]],
                                          xr_v.at[b], sem_g[b]).wait()

        def issue_scatters(j, pp):
            jj = j % SUP
            for b in (0, 1):
                @pl.when(pp == b)
                def _():
                    pltpu.make_async_copy(
                        we_v.at[b], acc_sh.at[dst_sup.at[jj]],
                        sem_s[b]).start(add=True)
                    pltpu.make_async_copy(
                        ex_v.at[pl.ds(pl.multiple_of(b * 48, 8), CH)],
                        den_sh.at[dst_sup.at[jj]], sem_s[b]).start(add=True)

        def wait_scatters(pp):
            for b in (0, 1):
                @pl.when(pp == b)
                def _():
                    pltpu.make_async_copy(
                        we_v.at[b], acc_sh.at[dst_sup.at[0]],
                        sem_s[b]).wait()
                    pltpu.make_async_copy(
                        ex_v.at[pl.ds(pl.multiple_of(b * 48, 8), CH)],
                        den_sh.at[dst_sup.at[0]], sem_s[b]).wait()

        def load_super(sidx):
            pltpu.sync_copy(src_hbm.at[wid, sidx], src_sup)
            pltpu.sync_copy(dst_hbm.at[wid, sidx], dst_sup)

        def compute(j, pp):
            for b in (0, 1):
                @pl.when(pp == b)
                def _():
                    exbase = pl.multiple_of(b * 48, 16)

                    def edge(e, exlane):
                        acc = jnp.zeros((L,), jnp.float32)
                        xr_regs = []
                        for kk in range(NSUB):
                            sl = pl.ds(kk * L, L)
                            r = xr_v[b, e, sl]
                            t = xl_v[b, e, sl] + we_v[b, e, sl] + r
                            lk = jnp.maximum(t, t * 0.2)
                            acc = acc + lk * att_k[kk]
                            xr_regs.append(r)
                        for sft in (1, 2, 4, 8):
                            acc = acc + lax.gather(
                                acc, (io ^ sft)[:, None], dimension_numbers=_DN,
                                slice_sizes=(1,),
                                mode=lax.GatherScatterMode.PROMISE_IN_BOUNDS)
                        exv = jnp.exp(acc)
                        for kk in range(NSUB):
                            we_v[b, e, pl.ds(kk * L, L)] = xr_regs[kk] * exv
                        exlane = jnp.where(io == e % L, exv, exlane)
                        ex_v[pl.ds(exbase + e // L * L, L)] = exlane
                        return exlane

                    lax.fori_loop(0, CH, edge, jnp.zeros((L,), jnp.float32))

        load_super(0)
        issue_gathers(0, 0)

        def body(j, carry):
            pp = j % 2
            wait_gathers(pp)
            compute(j, pp)

            @pl.when((j >= 1) & (j % SUP != 0))
            def _():
                wait_scatters(1 - pp)
            issue_scatters(j, pp)

            @pl.when(j < NCHUNK - 1)
            def _():
                @pl.when((j + 1) % SUP == 0)
                def _():
                    wait_scatters(pp)
                    load_super((j + 1) // SUP)
                issue_gathers(j + 1, 1 - pp)
            return carry

        lax.fori_loop(0, NCHUNK, body, 0)
        wait_scatters((NCHUNK - 1) % 2)

        plsc.subcore_barrier()

        @pl.when(sid == 0)
        def _():
            pltpu.sync_copy(acc_sh, acc_hbm.at[cid])
            pltpu.sync_copy(den_sh, den_hbm.at[cid])

    return edge_pass


def kernel(x, edge_index, edge_attr, c1_Wl, c1_Wr, c1_We, c1_att, c1_b,
           c2_Wl, c2_Wr, c2_We, c2_att, c2_b, lin_W, lin_b):
    src = edge_index[0].reshape(NW, NSUP, SUP, CH)
    dst = edge_index[1].reshape(NW, NSUP, SUP, CH)

    xl1, xr1 = _mm2(x, c1_Wl, c1_Wr, 1000)
    we1, we2 = _mm2(edge_attr, c1_We, c2_We, 2000)

    ep = _edge_pass()
    acc1, den1 = ep(xl1, xr1, we1, src, dst, c1_att)
    xl2, xr2 = _merge(acc1, den1, c1_b.reshape(1, D), c2_Wl, c2_Wr)
    acc2, den2 = ep(xl2, xr2, we2, src, dst, c2_att)
    return _final(acc2, den2, c2_b.reshape(1, D), lin_W, lin_b.reshape(1, D_OUT))


# speedup vs baseline: 7.6802x; 1.1352x over previous
import functools

import jax
import jax.numpy as jnp
from jax import lax
from jax.experimental import pallas as pl
from jax.experimental.pallas import tpu as pltpu
from jax.experimental.pallas import tpu_sc as plsc

N_NODES = 10000
E_EDGES = 320000
D = 128
D_OUT = 64
NC, NS, L = 2, 16, 16
NW = NC * NS
EPT = E_EDGES // NW
CH = 40
NCHUNK = EPT // CH
SUP = 10
NSUP = NCHUNK // SUP
ZROWS = 16
NSUB = D // L

_DN = lax.GatherDimensionNumbers(
    offset_dims=(), collapsed_slice_dims=(0,), start_index_map=(0,))


def _mm2_body(x_ref, wa_ref, wb_ref, oa_ref, ob_ref):
    x = x_ref[...]
    oa_ref[...] = jnp.dot(x, wa_ref[...], preferred_element_type=jnp.float32)
    ob_ref[...] = jnp.dot(x, wb_ref[...], preferred_element_type=jnp.float32)


def _mm2(x, wa, wb, blk):
    n, k = x.shape
    return pl.pallas_call(
        _mm2_body,
        grid=(n // blk,),
        in_specs=[
            pl.BlockSpec((blk, k), lambda i: (i, 0)),
            pl.BlockSpec(wa.shape, lambda i: (0, 0)),
            pl.BlockSpec(wb.shape, lambda i: (0, 0)),
        ],
        out_specs=[
            pl.BlockSpec((blk, wa.shape[1]), lambda i: (i, 0)),
            pl.BlockSpec((blk, wb.shape[1]), lambda i: (i, 0)),
        ],
        out_shape=[
            jax.ShapeDtypeStruct((n, wa.shape[1]), jnp.float32),
            jax.ShapeDtypeStruct((n, wb.shape[1]), jnp.float32),
        ],
    )(x, wa, wb)


def _merge_body(acc_ref, den_ref, b_ref, wl_ref, wr_ref, xl_ref, xr_ref):
    num = acc_ref[0] + acc_ref[1]
    i = pl.program_id(0)
    den = (den_ref[0, i, :] + den_ref[1, i, :])[:, None]
    h = num / (den + 1e-16) + b_ref[...]
    h = jnp.where(h > 0, h, jnp.exp(jnp.minimum(h, 0.0)) - 1.0)
    xl_ref[...] = jnp.dot(h, wl_ref[...], preferred_element_type=jnp.float32)
    xr_ref[...] = jnp.dot(h, wr_ref[...], preferred_element_type=jnp.float32)


def _merge(acc, den, b, wl, wr, blk=1000):
    return pl.pallas_call(
        _merge_body,
        grid=(N_NODES // blk,),
        in_specs=[
            pl.BlockSpec((NC, blk, D), lambda i: (0, i, 0)),
            pl.BlockSpec((NC, N_NODES // blk, blk), lambda i: (0, 0, 0)),
            pl.BlockSpec((1, D), lambda i: (0, 0)),
            pl.BlockSpec(wl.shape, lambda i: (0, 0)),
            pl.BlockSpec(wr.shape, lambda i: (0, 0)),
        ],
        out_specs=[
            pl.BlockSpec((blk, D), lambda i: (i, 0)),
            pl.BlockSpec((blk, D), lambda i: (i, 0)),
        ],
        out_shape=[
            jax.ShapeDtypeStruct((N_NODES, D), jnp.float32),
            jax.ShapeDtypeStruct((N_NODES, D), jnp.float32),
        ],
    )(acc, den.reshape(NC, N_NODES // blk, blk), b, wl, wr)


def _final_body(acc_ref, den_ref, b_ref, w_ref, lb_ref, o_ref):
    num = acc_ref[0] + acc_ref[1]
    i = pl.program_id(0)
    den = (den_ref[0, i, :] + den_ref[1, i, :])[:, None]
    h = num / (den + 1e-16) + b_ref[...]
    o_ref[...] = (
        jnp.dot(h, w_ref[...], preferred_element_type=jnp.float32) + lb_ref[...]
    )


def _final(acc, den, b, w, lb, blk=1000):
    return pl.pallas_call(
        _final_body,
        grid=(N_NODES // blk,),
        in_specs=[
            pl.BlockSpec((NC, blk, D), lambda i: (0, i, 0)),
            pl.BlockSpec((NC, N_NODES // blk, blk), lambda i: (0, 0, 0)),
            pl.BlockSpec((1, D), lambda i: (0, 0)),
            pl.BlockSpec(w.shape, lambda i: (0, 0)),
            pl.BlockSpec((1, D_OUT), lambda i: (0, 0)),
        ],
        out_specs=pl.BlockSpec((blk, D_OUT), lambda i: (i, 0)),
        out_shape=jax.ShapeDtypeStruct((N_NODES, D_OUT), jnp.float32),
    )(acc, den.reshape(NC, N_NODES // blk, blk), b, w, lb)


@functools.cache
def _edge_pass():
    mesh = plsc.VectorSubcoreMesh(core_axis_name="c", subcore_axis_name="s")

    @functools.partial(
        pl.kernel,
        out_type=(jax.ShapeDtypeStruct((NC, N_NODES, D), jnp.float32),
                  jax.ShapeDtypeStruct((NC, N_NODES), jnp.float32)),
        mesh=mesh,
        scratch_types=[
            pltpu.VMEM((SUP, CH), jnp.int32),
            pltpu.VMEM((SUP, CH), jnp.int32),
            pltpu.VMEM((2, CH, D), jnp.float32),
            pltpu.VMEM((2, CH, D), jnp.float32),
            pltpu.VMEM((2, CH, D), jnp.float32),
            pltpu.VMEM((D,), jnp.float32),
            pltpu.VMEM((ZROWS, D), jnp.float32),
            pltpu.VMEM((1024,), jnp.float32),
            pltpu.VMEM((96,), jnp.float32),
            pltpu.VMEM_SHARED((N_NODES, D), jnp.float32),
            pltpu.VMEM_SHARED((N_NODES,), jnp.float32),
            pltpu.SemaphoreType.DMA,
            pltpu.SemaphoreType.DMA,
            pltpu.SemaphoreType.DMA,
            pltpu.SemaphoreType.DMA,
        ],
    )
    def edge_pass(xl_hbm, xr_hbm, we_hbm, src_hbm, dst_hbm, att_hbm,
                  acc_hbm, den_hbm,
                  src_sup, dst_sup, we_v, xl_v, xr_v, att_v, z_v, zd_v, ex_v,
                  acc_sh, den_sh, sem_g0, sem_g1, sem_s0, sem_s1):
        cid = lax.axis_index("c")
        sid = lax.axis_index("s")
        wid = sid * NC + cid
        base = wid * EPT

        pltpu.sync_copy(att_hbm, att_v)

        zv = jnp.zeros((L,), jnp.float32)

        def zrow(i, c):
            z_v[i // NSUB, pl.ds((i % NSUB) * L, L)] = zv
            return c
        lax.fori_loop(0, ZROWS * NSUB, zrow, 0)

        def dzero(i, c):
            zd_v[pl.ds(i * L, L)] = zv
            return c
        lax.fori_loop(0, 1024 // L, dzero, 0)

        @pl.when(sid < N_NODES // 1000)
        def _():
            off = pl.multiple_of(sid * 1000, 1000)
            pltpu.sync_copy(zd_v.at[pl.ds(0, 1000)], den_sh.at[pl.ds(off, 1000)])

        nzc = N_NODES // ZROWS // NS
        zrem = (N_NODES // ZROWS) % NS

        def zfire(t, c):
            m = t * NS + sid
            off = pl.multiple_of(m * ZROWS, ZROWS)
            pltpu.make_async_copy(z_v, acc_sh.at[pl.ds(off, ZROWS)],
                                  sem_g0).start()
            return c
        lax.fori_loop(0, nzc, zfire, 0)

        @pl.when(sid < zrem)
        def _():
            m = nzc * NS + sid
            off = pl.multiple_of(m * ZROWS, ZROWS)
            pltpu.make_async_copy(z_v, acc_sh.at[pl.ds(off, ZROWS)],
                                  sem_g0).start()

        def zdrain(t, c):
            pltpu.make_async_copy(
                z_v, acc_sh.at[pl.ds(pl.multiple_of(0, ZROWS), ZROWS)],
                sem_g0).wait()
            return c
        lax.fori_loop(0, nzc, zdrain, 0)

        @pl.when(sid < zrem)
        def _():
            pltpu.make_async_copy(
                z_v, acc_sh.at[pl.ds(pl.multiple_of(0, ZROWS), ZROWS)],
                sem_g0).wait()
        plsc.subcore_barrier()

        att_k = [att_v[pl.ds(kk * L, L)] for kk in range(NSUB)]
        io = lax.iota(jnp.int32, L)
        sem_g = [sem_g0, sem_g1]
        sem_s = [sem_s0, sem_s1]

        def issue_gathers(j, pp):
            jj = j % SUP
            off = pl.multiple_of(base + j * CH, CH)
            for b in (0, 1):
                @pl.when(pp == b)
                def _():
                    pltpu.make_async_copy(we_hbm.at[pl.ds(off, CH)],
                                          we_v.at[b], sem_g[b]).start()
                    pltpu.make_async_copy(xl_hbm.at[dst_sup.at[jj]],
                                          xl_v.at[b], sem_g[b]).start()
                    pltpu.make_async_copy(xr_hbm.at[src_sup.at[jj]],
                                          xr_v.at[b], sem_g[b]).start()

        def wait_gathers(pp):
            for b in (0, 1):
                @pl.when(pp == b)
                def _():
                    off0 = pl.multiple_of(0, CH)
                    pltpu.make_async_copy(we_hbm.at[pl.ds(off0, CH)],
                                          we_v.at[b], sem_g[b]).wait()
                    pltpu.make_async_copy(xl_hbm.at[dst_sup.at[0]],
                                          xl_v.at[b], sem_g[b]).wait()
                    pltpu.make_async_copy(xr_hbm.at[src_sup.at[0]],
                                          xr_v.at[b], sem_g[b]).wait()

        def issue_scatters(j, pp):
            jj = j % SUP
            for b in (0, 1):
                @pl.when(pp == b)
                def _():
                    pltpu.make_async_copy(
                        we_v.at[b], acc_sh.at[dst_sup.at[jj]],
                        sem_s[b]).start(add=True)
                    pltpu.make_async_copy(
                        ex_v.at[pl.ds(pl.multiple_of(b * 48, 8), CH)],
                        den_sh.at[dst_sup.at[jj]], sem_s[b]).start(add=True)

        def wait_scatters(pp):
            for b in (0, 1):
                @pl.when(pp == b)
                def _():
                    pltpu.make_async_copy(
                        we_v.at[b], acc_sh.at[dst_sup.at[0]],
                        sem_s[b]).wait()
                    pltpu.make_async_copy(
                        ex_v.at[pl.ds(pl.multiple_of(b * 48, 8), CH)],
                        den_sh.at[dst_sup.at[0]], sem_s[b]).wait()

        def load_super(sidx):
            pltpu.sync_copy(src_hbm.at[wid, sidx], src_sup)
            pltpu.sync_copy(dst_hbm.at[wid, sidx], dst_sup)

        def compute(j, pp):
            for b in (0, 1):
                @pl.when(pp == b)
                def _():
                    exbase = pl.multiple_of(b * 48, 16)

                    def edge_pair(i, exlane):
                        exvs = []
                        regs = []
                        for e in (2 * i, 2 * i + 1):
                            acc0 = jnp.zeros((L,), jnp.float32)
                            acc1 = jnp.zeros((L,), jnp.float32)
                            xr_regs = []
                            for kk in range(NSUB):
                                sl = pl.ds(kk * L, L)
                                r = xr_v[b, e, sl]
                                t = xl_v[b, e, sl] + we_v[b, e, sl] + r
                                lk = jnp.maximum(t, t * 0.2)
                                if kk % 2 == 0:
                                    acc0 = acc0 + lk * att_k[kk]
                                else:
                                    acc1 = acc1 + lk * att_k[kk]
                                xr_regs.append(r)
                            acc = acc0 + acc1
                            for sft in (1, 2, 4, 8):
                                acc = acc + lax.gather(
                                    acc, (io ^ sft)[:, None],
                                    dimension_numbers=_DN, slice_sizes=(1,),
                                    mode=lax.GatherScatterMode.PROMISE_IN_BOUNDS)
                            exvs.append(jnp.exp(acc))
                            regs.append(xr_regs)
                        for q, e in ((0, 2 * i), (1, 2 * i + 1)):
                            for kk in range(NSUB):
                                we_v[b, e, pl.ds(kk * L, L)] = regs[q][kk] * exvs[q]
                        e0 = 2 * i
                        exlane = jnp.where(io == e0 % L, exvs[0], exlane)
                        exlane = jnp.where(io == (e0 + 1) % L, exvs[1], exlane)
                        ex_v[pl.ds(exbase + e0 // L * L, L)] = exlane
                        return exlane

                    lax.fori_loop(0, CH // 2, edge_pair,
                                  jnp.zeros((L,), jnp.float32))

        load_super(0)
        issue_gathers(0, 0)

        def body(j, carry):
            pp = j % 2
            wait_gathers(pp)
            compute(j, pp)

            @pl.when((j >= 1) & (j % SUP != 0))
            def _():
                wait_scatters(1 - pp)
            issue_scatters(j, pp)

            @pl.when(j < NCHUNK - 1)
            def _():
                @pl.when((j + 1) % SUP == 0)
                def _():
                    wait_scatters(pp)
                    load_super((j + 1) // SUP)
                issue_gathers(j + 1, 1 - pp)
            return carry

        lax.fori_loop(0, NCHUNK, body, 0)
        wait_scatters((NCHUNK - 1) % 2)

        plsc.subcore_barrier()

        @pl.when(sid == 0)
        def _():
            pltpu.sync_copy(acc_sh, acc_hbm.at[cid])
            pltpu.sync_copy(den_sh, den_hbm.at[cid])

    return edge_pass


def kernel(x, edge_index, edge_attr, c1_Wl, c1_Wr, c1_We, c1_att, c1_b,
           c2_Wl, c2_Wr, c2_We, c2_att, c2_b, lin_W, lin_b):
    src = edge_index[0].reshape(NW, NSUP, SUP, CH)
    dst = edge_index[1].reshape(NW, NSUP, SUP, CH)

    xl1, xr1 = _mm2(x, c1_Wl, c1_Wr, 1000)
    we1, we2 = _mm2(edge_attr, c1_We, c2_We, 2000)

    ep = _edge_pass()
    acc1, den1 = ep(xl1, xr1, we1, src, dst, c1_att)
    xl2, xr2 = _merge(acc1, den1, c1_b.reshape(1, D), c2_Wl, c2_Wr)
    acc2, den2 = ep(xl2, xr2, we2, src, dst, c2_att)
    return _final(acc2, den2, c2_b.reshape(1, D), lin_W, lin_b.reshape(1, D_OUT))


# speedup vs baseline: 7.9787x; 1.0389x over previous
import functools

import jax
import jax.numpy as jnp
from jax import lax
from jax.experimental import pallas as pl
from jax.experimental.pallas import tpu as pltpu
from jax.experimental.pallas import tpu_sc as plsc

N_NODES = 10000
E_EDGES = 320000
D = 128
D_OUT = 64
NC, NS, L = 2, 16, 16
NW = NC * NS
EPT = E_EDGES // NW
CH = 40
NCHUNK = EPT // CH
SUP = 10
NSUP = NCHUNK // SUP
ZROWS = 16
NSUB = D // L

_DN = lax.GatherDimensionNumbers(
    offset_dims=(), collapsed_slice_dims=(0,), start_index_map=(0,))


def _mm2_body(x_ref, wa_ref, wb_ref, oa_ref, ob_ref):
    x = x_ref[...]
    oa_ref[...] = jnp.dot(x, wa_ref[...], preferred_element_type=jnp.float32)
    ob_ref[...] = jnp.dot(x, wb_ref[...], preferred_element_type=jnp.float32)


def _mm2(x, wa, wb, blk):
    n, k = x.shape
    return pl.pallas_call(
        _mm2_body,
        grid=(n // blk,),
        in_specs=[
            pl.BlockSpec((blk, k), lambda i: (i, 0)),
            pl.BlockSpec(wa.shape, lambda i: (0, 0)),
            pl.BlockSpec(wb.shape, lambda i: (0, 0)),
        ],
        out_specs=[
            pl.BlockSpec((blk, wa.shape[1]), lambda i: (i, 0)),
            pl.BlockSpec((blk, wb.shape[1]), lambda i: (i, 0)),
        ],
        out_shape=[
            jax.ShapeDtypeStruct((n, wa.shape[1]), jnp.float32),
            jax.ShapeDtypeStruct((n, wb.shape[1]), jnp.float32),
        ],
    )(x, wa, wb)


def _merge_body(acc_ref, den_ref, b_ref, wl_ref, wr_ref, xl_ref, xr_ref):
    num = acc_ref[0] + acc_ref[1]
    i = pl.program_id(0)
    den = (den_ref[0, i, :] + den_ref[1, i, :])[:, None]
    h = num / (den + 1e-16) + b_ref[...]
    h = jnp.where(h > 0, h, jnp.exp(jnp.minimum(h, 0.0)) - 1.0)
    xl_ref[...] = jnp.dot(h, wl_ref[...], preferred_element_type=jnp.float32)
    xr_ref[...] = jnp.dot(h, wr_ref[...], preferred_element_type=jnp.float32)


def _merge(acc, den, b, wl, wr, blk=1000):
    return pl.pallas_call(
        _merge_body,
        grid=(N_NODES // blk,),
        in_specs=[
            pl.BlockSpec((NC, blk, D), lambda i: (0, i, 0)),
            pl.BlockSpec((NC, N_NODES // blk, blk), lambda i: (0, 0, 0)),
            pl.BlockSpec((1, D), lambda i: (0, 0)),
            pl.BlockSpec(wl.shape, lambda i: (0, 0)),
            pl.BlockSpec(wr.shape, lambda i: (0, 0)),
        ],
        out_specs=[
            pl.BlockSpec((blk, D), lambda i: (i, 0)),
            pl.BlockSpec((blk, D), lambda i: (i, 0)),
        ],
        out_shape=[
            jax.ShapeDtypeStruct((N_NODES, D), jnp.float32),
            jax.ShapeDtypeStruct((N_NODES, D), jnp.float32),
        ],
    )(acc, den.reshape(NC, N_NODES // blk, blk), b, wl, wr)


def _final_body(acc_ref, den_ref, b_ref, w_ref, lb_ref, o_ref):
    num = acc_ref[0] + acc_ref[1]
    i = pl.program_id(0)
    den = (den_ref[0, i, :] + den_ref[1, i, :])[:, None]
    h = num / (den + 1e-16) + b_ref[...]
    o_ref[...] = (
        jnp.dot(h, w_ref[...], preferred_element_type=jnp.float32) + lb_ref[...]
    )


def _final(acc, den, b, w, lb, blk=1000):
    return pl.pallas_call(
        _final_body,
        grid=(N_NODES // blk,),
        in_specs=[
            pl.BlockSpec((NC, blk, D), lambda i: (0, i, 0)),
            pl.BlockSpec((NC, N_NODES // blk, blk), lambda i: (0, 0, 0)),
            pl.BlockSpec((1, D), lambda i: (0, 0)),
            pl.BlockSpec(w.shape, lambda i: (0, 0)),
            pl.BlockSpec((1, D_OUT), lambda i: (0, 0)),
        ],
        out_specs=pl.BlockSpec((blk, D_OUT), lambda i: (i, 0)),
        out_shape=jax.ShapeDtypeStruct((N_NODES, D_OUT), jnp.float32),
    )(acc, den.reshape(NC, N_NODES // blk, blk), b, w, lb)


@functools.cache
def _edge_pass():
    mesh = plsc.VectorSubcoreMesh(core_axis_name="c", subcore_axis_name="s")

    @functools.partial(
        pl.kernel,
        out_type=(jax.ShapeDtypeStruct((NC, N_NODES, D), jnp.float32),
                  jax.ShapeDtypeStruct((NC, N_NODES), jnp.float32)),
        mesh=mesh,
        scratch_types=[
            pltpu.VMEM((2, SUP, CH), jnp.int32),
            pltpu.VMEM((2, SUP, CH), jnp.int32),
            pltpu.VMEM((2, CH, D), jnp.float32),
            pltpu.VMEM((2, CH, D), jnp.float32),
            pltpu.VMEM((2, CH, D), jnp.float32),
            pltpu.VMEM((D,), jnp.float32),
            pltpu.VMEM((ZROWS, D), jnp.float32),
            pltpu.VMEM((1024,), jnp.float32),
            pltpu.VMEM((96,), jnp.float32),
            pltpu.VMEM_SHARED((N_NODES, D), jnp.float32),
            pltpu.VMEM_SHARED((N_NODES,), jnp.float32),
            pltpu.SemaphoreType.DMA,
            pltpu.SemaphoreType.DMA,
            pltpu.SemaphoreType.DMA,
            pltpu.SemaphoreType.DMA,
            pltpu.SemaphoreType.DMA,
        ],
    )
    def edge_pass(xl_hbm, xr_hbm, we_hbm, src_hbm, dst_hbm, att_hbm,
                  acc_hbm, den_hbm,
                  src_sup, dst_sup, we_v, xl_v, xr_v, att_v, z_v, zd_v, ex_v,
                  acc_sh, den_sh, sem_g0, sem_g1, sem_s0, sem_s1, sem_i):
        cid = lax.axis_index("c")
        sid = lax.axis_index("s")
        wid = sid * NC + cid
        base = wid * EPT

        pltpu.sync_copy(att_hbm, att_v)

        zv = jnp.zeros((L,), jnp.float32)

        def zrow(i, c):
            z_v[i // NSUB, pl.ds((i % NSUB) * L, L)] = zv
            return c
        lax.fori_loop(0, ZROWS * NSUB, zrow, 0)

        def dzero(i, c):
            zd_v[pl.ds(i * L, L)] = zv
            return c
        lax.fori_loop(0, 1024 // L, dzero, 0)

        @pl.when(sid < N_NODES // 1000)
        def _():
            off = pl.multiple_of(sid * 1000, 1000)
            pltpu.sync_copy(zd_v.at[pl.ds(0, 1000)], den_sh.at[pl.ds(off, 1000)])

        nzc = N_NODES // ZROWS // NS
        zrem = (N_NODES // ZROWS) % NS

        def zfire(t, c):
            m = t * NS + sid
            off = pl.multiple_of(m * ZROWS, ZROWS)
            pltpu.make_async_copy(z_v, acc_sh.at[pl.ds(off, ZROWS)],
                                  sem_g0).start()
            return c
        lax.fori_loop(0, nzc, zfire, 0)

        @pl.when(sid < zrem)
        def _():
            m = nzc * NS + sid
            off = pl.multiple_of(m * ZROWS, ZROWS)
            pltpu.make_async_copy(z_v, acc_sh.at[pl.ds(off, ZROWS)],
                                  sem_g0).start()

        def zdrain(t, c):
            pltpu.make_async_copy(
                z_v, acc_sh.at[pl.ds(pl.multiple_of(0, ZROWS), ZROWS)],
                sem_g0).wait()
            return c
        lax.fori_loop(0, nzc, zdrain, 0)

        @pl.when(sid < zrem)
        def _():
            pltpu.make_async_copy(
                z_v, acc_sh.at[pl.ds(pl.multiple_of(0, ZROWS), ZROWS)],
                sem_g0).wait()
        plsc.subcore_barrier()

        att_k = [att_v[pl.ds(kk * L, L)] for kk in range(NSUB)]
        io = lax.iota(jnp.int32, L)
        sem_g = [sem_g0, sem_g1]
        sem_s = [sem_s0, sem_s1]

        def issue_gathers(j, pp, sb):
            jj = j % SUP
            off = pl.multiple_of(base + j * CH, CH)
            for b in (0, 1):
                @pl.when(pp == b)
                def _():
                    pltpu.make_async_copy(we_hbm.at[pl.ds(off, CH)],
                                          we_v.at[b], sem_g[b]).start()
                    for sbb in (0, 1):
                        @pl.when(sb == sbb)
                        def _():
                            pltpu.make_async_copy(
                                xl_hbm.at[dst_sup.at[sbb].at[jj]],
                                xl_v.at[b], sem_g[b]).start()
                            pltpu.make_async_copy(
                                xr_hbm.at[src_sup.at[sbb].at[jj]],
                                xr_v.at[b], sem_g[b]).start()

        def wait_gathers(pp):
            for b in (0, 1):
                @pl.when(pp == b)
                def _():
                    off0 = pl.multiple_of(0, CH)
                    pltpu.make_async_copy(we_hbm.at[pl.ds(off0, CH)],
                                          we_v.at[b], sem_g[b]).wait()
                    pltpu.make_async_copy(xl_hbm.at[dst_sup.at[0].at[0]],
                                          xl_v.at[b], sem_g[b]).wait()
                    pltpu.make_async_copy(xr_hbm.at[src_sup.at[0].at[0]],
                                          xr_v.at[b], sem_g[b]).wait()

        def issue_scatters(j, pp, sb):
            jj = j % SUP
            for b in (0, 1):
                @pl.when(pp == b)
                def _():
                    for sbb in (0, 1):
                        @pl.when(sb == sbb)
                        def _():
                            pltpu.make_async_copy(
                                we_v.at[b],
                                acc_sh.at[dst_sup.at[sbb].at[jj]],
                                sem_s[b]).start(add=True)
                            pltpu.make_async_copy(
                                ex_v.at[pl.ds(pl.multiple_of(b * 48, 8), CH)],
                                den_sh.at[dst_sup.at[sbb].at[jj]],
                                sem_s[b]).start(add=True)

        def wait_scatters(pp):
            for b in (0, 1):
                @pl.when(pp == b)
                def _():
                    pltpu.make_async_copy(
                        we_v.at[b], acc_sh.at[dst_sup.at[0].at[0]],
                        sem_s[b]).wait()
                    pltpu.make_async_copy(
                        ex_v.at[pl.ds(pl.multiple_of(b * 48, 8), CH)],
                        den_sh.at[dst_sup.at[0].at[0]], sem_s[b]).wait()

        def issue_super(sidx, sbuf):
            for sbb in (0, 1):
                @pl.when(sbuf == sbb)
                def _():
                    pltpu.make_async_copy(src_hbm.at[wid, sidx],
                                          src_sup.at[sbb], sem_i).start()
                    pltpu.make_async_copy(dst_hbm.at[wid, sidx],
                                          dst_sup.at[sbb], sem_i).start()

        def wait_super():
            pltpu.make_async_copy(src_hbm.at[wid, 0], src_sup.at[0],
                                  sem_i).wait()
            pltpu.make_async_copy(dst_hbm.at[wid, 0], dst_sup.at[0],
                                  sem_i).wait()

        def compute(j, pp):
            for b in (0, 1):
                @pl.when(pp == b)
                def _():
                    exbase = pl.multiple_of(b * 48, 16)

                    def edge_pair(i, exlane):
                        exvs = []
                        regs = []
                        for e in (2 * i, 2 * i + 1):
                            acc0 = jnp.zeros((L,), jnp.float32)
                            acc1 = jnp.zeros((L,), jnp.float32)
                            xr_regs = []
                            for kk in range(NSUB):
                                sl = pl.ds(kk * L, L)
                                r = xr_v[b, e, sl]
                                t = xl_v[b, e, sl] + we_v[b, e, sl] + r
                                lk = jnp.maximum(t, t * 0.2)
                                if kk % 2 == 0:
                                    acc0 = acc0 + lk * att_k[kk]
                                else:
                                    acc1 = acc1 + lk * att_k[kk]
                                xr_regs.append(r)
                            acc = acc0 + acc1
                            for sft in (1, 2, 4, 8):
                                acc = acc + lax.gather(
                                    acc, (io ^ sft)[:, None],
                                    dimension_numbers=_DN, slice_sizes=(1,),
                                    mode=lax.GatherScatterMode.PROMISE_IN_BOUNDS)
                            exvs.append(jnp.exp(acc))
                            regs.append(xr_regs)
                        for q, e in ((0, 2 * i), (1, 2 * i + 1)):
                            for kk in range(NSUB):
                                we_v[b, e, pl.ds(kk * L, L)] = regs[q][kk] * exvs[q]
                        e0 = 2 * i
                        exlane = jnp.where(io == e0 % L, exvs[0], exlane)
                        exlane = jnp.where(io == (e0 + 1) % L, exvs[1], exlane)
                        ex_v[pl.ds(exbase + e0 // L * L, L)] = exlane
                        return exlane

                    lax.fori_loop(0, CH // 2, edge_pair,
                                  jnp.zeros((L,), jnp.float32))

        issue_super(0, 0)
        wait_super()
        issue_gathers(0, 0, 0)

        def body(j, carry):
            pp = j % 2
            sb = (j // SUP) % 2
            wait_gathers(pp)
            compute(j, pp)

            @pl.when(j >= 1)
            def _():
                wait_scatters(1 - pp)
            issue_scatters(j, pp, sb)

            @pl.when((j % SUP == 1) & (j // SUP + 1 < NSUP))
            def _():
                issue_super(j // SUP + 1, 1 - sb)

            @pl.when(j < NCHUNK - 1)
            def _():
                @pl.when((j + 1) % SUP == 0)
                def _():
                    wait_super()
                issue_gathers(j + 1, 1 - pp, ((j + 1) // SUP) % 2)
            return carry

        lax.fori_loop(0, NCHUNK, body, 0)
        wait_scatters((NCHUNK - 1) % 2)

        plsc.subcore_barrier()

        @pl.when(sid == 0)
        def _():
            pltpu.sync_copy(acc_sh, acc_hbm.at[cid])
            pltpu.sync_copy(den_sh, den_hbm.at[cid])

    return edge_pass


def kernel(x, edge_index, edge_attr, c1_Wl, c1_Wr, c1_We, c1_att, c1_b,
           c2_Wl, c2_Wr, c2_We, c2_att, c2_b, lin_W, lin_b):
    src = edge_index[0].reshape(NW, NSUP, SUP, CH)
    dst = edge_index[1].reshape(NW, NSUP, SUP, CH)

    xl1, xr1 = _mm2(x, c1_Wl, c1_Wr, 1000)
    we1, we2 = _mm2(edge_attr, c1_We, c2_We, 2000)

    ep = _edge_pass()
    acc1, den1 = ep(xl1, xr1, we1, src, dst, c1_att)
    xl2, xr2 = _merge(acc1, den1, c1_b.reshape(1, D), c2_Wl, c2_Wr)
    acc2, den2 = ep(xl2, xr2, we2, src, dst, c2_att)
    return _final(acc2, den2, c2_b.reshape(1, D), lin_W, lin_b.reshape(1, D_OUT))
